# Initial kernel scaffold; baseline (speedup 1.0000x reference)
import jax
import jax.numpy as jnp
from jax.experimental import pallas as pl


def kernel(root_tokens1, child_tokens1, child_parent1, root_tokens2, child_tokens2, child_parent2, embedding, W_c, b_c, W_ih_f, W_hh_f, b_ih_f, b_hh_f, W_ih_b, W_hh_b, b_ih_b, b_hh_b, W_out, b_out):
    raise NotImplementedError("write your pallas kernel here")


# speedup vs baseline: 1.7128x; 1.7128x over previous
import functools

import jax
import jax.numpy as jnp
from jax import lax
from jax.experimental import pallas as pl
from jax.experimental.pallas import tpu as pltpu
from jax.experimental.pallas import tpu_sc as plsc

S = 3200
NC = 51200
B = 64
L = 50
E = 128
H = 128

N_WORKERS = 32
SEG_PER_W = S // N_WORKERS
SEG_PAD = 104
ROOT0 = 2 * NC
N_ALL = 2 * NC + 2 * N_WORKERS * SEG_PAD
G_ROWS = N_ALL // N_WORKERS
G_CHUNK = 128
G_FULL = G_ROWS // G_CHUNK
G_TAIL = G_ROWS - G_FULL * G_CHUNK
R_CHUNK = 128

def _mesh():
    return plsc.VectorSubcoreMesh(
        core_axis_name="c", subcore_axis_name="s", num_cores=2, num_subcores=16
    )


def _worker_id():
    return lax.axis_index("s") * 2 + lax.axis_index("c")


def _gather_body(emb_hbm, tok_hbm, out_hbm, idx_v, rows_v, idx_t, rows_t, sem):
    base = _worker_id() * G_ROWS

    def chunk(i, _):
        start = base + i * G_CHUNK
        pltpu.sync_copy(tok_hbm.at[pl.ds(start, G_CHUNK)], idx_v)
        pltpu.async_copy(emb_hbm.at[idx_v], rows_v, sem).wait()
        pltpu.sync_copy(rows_v, out_hbm.at[pl.ds(start, G_CHUNK)])
        return 0

    lax.fori_loop(0, G_FULL, chunk, 0)
    tstart = base + G_FULL * G_CHUNK
    pltpu.sync_copy(tok_hbm.at[pl.ds(tstart, G_TAIL)], idx_t)
    pltpu.async_copy(emb_hbm.at[idx_t], rows_t, sem).wait()
    pltpu.sync_copy(rows_t, out_hbm.at[pl.ds(tstart, G_TAIL)])


def _gather(emb, tokens):
    return pl.kernel(
        _gather_body,
        out_type=jax.ShapeDtypeStruct((N_ALL, E), jnp.float32),
        mesh=_mesh(),
        scratch_types=[
            pltpu.VMEM((G_CHUNK,), jnp.int32),
            pltpu.VMEM((G_CHUNK, E), jnp.float32),
            pltpu.VMEM((G_TAIL,), jnp.int32),
            pltpu.VMEM((G_TAIL, E), jnp.float32),
            pltpu.SemaphoreType.DMA,
        ],
    )(emb, tokens)


def _mm_body(x_ref, w_ref, b_ref, o_ref):
    o_ref[...] = (
        jnp.dot(x_ref[...], w_ref[...], preferred_element_type=jnp.float32)
        + b_ref[...]
    )


def _combine_matmul(rows, W_c, b_c):
    blk = 256
    return pl.pallas_call(
        _mm_body,
        grid=(N_ALL // blk,),
        in_specs=[
            pl.BlockSpec((blk, E), lambda i: (i, 0)),
            pl.BlockSpec((E, E), lambda i: (0, 0)),
            pl.BlockSpec((1, E), lambda i: (0, 0)),
        ],
        out_specs=pl.BlockSpec((blk, E), lambda i: (i, 0)),
        out_shape=jax.ShapeDtypeStruct((N_ALL, E), jnp.float32),
    )(rows, W_c, b_c.reshape(1, E))


def _reduce_body(base_hbm, cp_hbm, woff_hbm, zero_hbm, out_hbm,
                 woff_v, idx_v, rows_v, root_v, acc_s, acc_m):
    wid = _worker_id()
    pltpu.sync_copy(woff_hbm, woff_v)
    seg_lo = wid * SEG_PER_W

    for side in range(2):
        crow0 = side * NC
        cp0 = side * NC
        blk = side * N_WORKERS + wid

        pltpu.sync_copy(zero_hbm, acc_s)
        pltpu.sync_copy(zero_hbm, acc_m)
        pltpu.sync_copy(
            base_hbm.at[pl.ds(ROOT0 + blk * SEG_PAD, SEG_PAD)], root_v
        )

        bvec = woff_v[pl.ds((side * N_WORKERS + wid) * 16, 16)]
        lo = bvec[0]
        hi = bvec[1]
        c0 = lo // R_CHUNK
        c1 = (hi + R_CHUNK - 1) // R_CHUNK

        def do_chunk(c, _):
            pltpu.sync_copy(cp_hbm.at[pl.ds(cp0 + c * R_CHUNK, R_CHUNK)], idx_v)
            pltpu.sync_copy(
                base_hbm.at[pl.ds(crow0 + c * R_CHUNK, R_CHUNK)], rows_v
            )

            def do_group(g, _):
                iv = idx_v[pl.ds(g * 16, 16)]
                for i in range(16):
                    ls = iv[i] - seg_lo

                    @pl.when(jnp.logical_and(ls >= 0, ls < SEG_PER_W))
                    def _(ls=ls, g=g, i=i):
                        r = g * 16 + i
                        for j in range(E // 16):
                            sl = pl.ds(j * 16, 16)
                            v = rows_v[r, sl]
                            acc_s[ls, sl] = acc_s[ls, sl] + v
                            acc_m[ls, sl] = jnp.maximum(acc_m[ls, sl], v)

                return 0

            lax.fori_loop(0, R_CHUNK // 16, do_group, 0)
            return 0

        lax.fori_loop(c0, c1, do_chunk, 0)

        def finalize(i, _):
            for j in range(E // 16):
                sl = pl.ds(j * 16, 16)
                acc_s[i, sl] = jnp.maximum(
                    root_v[i, sl] + acc_s[i, sl], acc_m[i, sl]
                )
            return 0

        lax.fori_loop(0, SEG_PER_W, finalize, 0)
        pltpu.sync_copy(acc_s, out_hbm.at[pl.ds(blk * SEG_PAD, SEG_PAD)])


def _reduce(base, cp, woff, zero):
    return pl.kernel(
        _reduce_body,
        out_type=jax.ShapeDtypeStruct((2 * N_WORKERS * SEG_PAD, E), jnp.float32),
        mesh=_mesh(),
        scratch_types=[
            pltpu.VMEM((2 * N_WORKERS * 16,), jnp.int32),
            pltpu.VMEM((R_CHUNK,), jnp.int32),
            pltpu.VMEM((R_CHUNK, E), jnp.float32),
            pltpu.VMEM((SEG_PAD, E), jnp.float32),
            pltpu.VMEM((SEG_PAD, E), jnp.float32),
            pltpu.VMEM((SEG_PAD, E), jnp.float32),
        ],
    )(base, cp, woff, zero)


def _gru_gate(gi, gh, h):
    r = jax.nn.sigmoid(gi[:, :H] + gh[:, :H])
    z = jax.nn.sigmoid(gi[:, H:2 * H] + gh[:, H:2 * H])
    n = jnp.tanh(gi[:, 2 * H:] + r * gh[:, 2 * H:])
    return (1.0 - z) * n + z * h


def _gru_body(x_ref, wif_ref, whf_ref, bif_ref, bhf_ref,
              wib_ref, whb_ref, bib_ref, bhb_ref, wo_ref, bo_ref,
              o_ref, gif_ref, gib_ref, hsf_ref):
    x = x_ref[...]
    gif_ref[...] = (
        jnp.dot(x, wif_ref[...], preferred_element_type=jnp.float32)
        + bif_ref[...]
    )
    gib_ref[...] = (
        jnp.dot(x, wib_ref[...], preferred_element_type=jnp.float32)
        + bib_ref[...]
    )
    whf = whf_ref[...]
    bhf = bhf_ref[...]

    def fstep(t, h):
        gi = gif_ref[pl.ds(t * 128, 128), :]
        gh = jnp.dot(h, whf, preferred_element_type=jnp.float32) + bhf
        h2 = _gru_gate(gi, gh, h)
        hsf_ref[pl.ds(t * 128, 128), :] = h2
        return h2

    lax.fori_loop(0, L, fstep, jnp.zeros((128, H), jnp.float32))

    whb = whb_ref[...]
    bhb = bhb_ref[...]

    def bstep(k, carry):
        h, m = carry
        t = L - 1 - k
        gi = gib_ref[pl.ds(t * 128, 128), :]
        gh = jnp.dot(h, whb, preferred_element_type=jnp.float32) + bhb
        h2 = _gru_gate(gi, gh, h)
        comb = hsf_ref[pl.ds(t * 128, 128), :] + h2
        return h2, jnp.maximum(m, comb)

    _, m = lax.fori_loop(
        0, L,
        bstep,
        (jnp.zeros((128, H), jnp.float32),
         jnp.full((128, H), -jnp.inf, jnp.float32)),
    )

    d = jnp.abs(m[:B, :] - m[B:, :])
    logits = jnp.sum(d * wo_ref[...], axis=1, keepdims=True) + bo_ref[...]
    o_ref[...] = jax.nn.sigmoid(logits)


def _gru_head(x2d, W_ih_f, W_hh_f, b_ih_f, b_hh_f,
              W_ih_b, W_hh_b, b_ih_b, b_hh_b, W_out, b_out):
    return pl.pallas_call(
        _gru_body,
        out_shape=jax.ShapeDtypeStruct((B, 1), jnp.float32),
        scratch_shapes=[
            pltpu.VMEM((L * 128, 3 * H), jnp.float32),
            pltpu.VMEM((L * 128, 3 * H), jnp.float32),
            pltpu.VMEM((L * 128, H), jnp.float32),
        ],
    )(x2d, W_ih_f, W_hh_f, b_ih_f.reshape(1, 3 * H), b_hh_f.reshape(1, 3 * H),
      W_ih_b, W_hh_b, b_ih_b.reshape(1, 3 * H), b_hh_b.reshape(1, 3 * H),
      W_out.reshape(1, H), b_out.reshape(1, 1))


def kernel(root_tokens1, child_tokens1, child_parent1,
           root_tokens2, child_tokens2, child_parent2,
           embedding, W_c, b_c,
           W_ih_f, W_hh_f, b_ih_f, b_hh_f,
           W_ih_b, W_hh_b, b_ih_b, b_hh_b,
           W_out, b_out):
    root_pad = (
        jnp.zeros((2, N_WORKERS, SEG_PAD), jnp.int32)
        .at[0, :, :SEG_PER_W].set(
            root_tokens1.astype(jnp.int32).reshape(N_WORKERS, SEG_PER_W))
        .at[1, :, :SEG_PER_W].set(
            root_tokens2.astype(jnp.int32).reshape(N_WORKERS, SEG_PER_W))
        .reshape(-1)
    )
    tokens = jnp.concatenate([
        child_tokens1.astype(jnp.int32), child_tokens2.astype(jnp.int32),
        root_pad,
    ])
    cp1 = child_parent1.astype(jnp.int32)
    cp2 = child_parent2.astype(jnp.int32)
    cp = jnp.concatenate([cp1, cp2])

    bounds = jnp.arange(0, S + 1, SEG_PER_W)
    offs1 = jnp.searchsorted(cp1, bounds).astype(jnp.int32)
    offs2 = jnp.searchsorted(cp2, bounds).astype(jnp.int32)
    woff = (
        jnp.zeros((2, N_WORKERS, 16), jnp.int32)
        .at[0, :, 0].set(offs1[:-1]).at[0, :, 1].set(offs1[1:])
        .at[1, :, 0].set(offs2[:-1]).at[1, :, 1].set(offs2[1:])
        .reshape(-1)
    )

    rows = _gather(embedding, tokens)
    base = _combine_matmul(rows, W_c, b_c)
    out_p = _reduce(base, cp, woff, jnp.zeros((SEG_PAD, E), jnp.float32))
    stmt = out_p.reshape(2, N_WORKERS, SEG_PAD, E)[:, :, :SEG_PER_W, :]

    x = jnp.transpose(stmt.reshape(2, B, L, E), (2, 0, 1, 3))
    x2d = x.reshape(L * 2 * B, E)

    return _gru_head(x2d, W_ih_f, W_hh_f, b_ih_f, b_hh_f,
                     W_ih_b, W_hh_b, b_ih_b, b_hh_b, W_out, b_out)


# speedup vs baseline: 1.7732x; 1.0352x over previous
import functools

import jax
import jax.numpy as jnp
from jax import lax
from jax.experimental import pallas as pl
from jax.experimental.pallas import tpu as pltpu
from jax.experimental.pallas import tpu_sc as plsc

S = 3200
NC = 51200
B = 64
L = 50
E = 128
H = 128

N_WORKERS = 32
SEG_PER_W = S // N_WORKERS
SEG_PAD = 104
ROOT0 = 2 * NC
N_ALL = 2 * NC + 2 * N_WORKERS * SEG_PAD
G_ROWS = N_ALL // N_WORKERS
G_CHUNK = 128
G_FULL = G_ROWS // G_CHUNK
G_TAIL = G_ROWS - G_FULL * G_CHUNK
R_CHUNK = 128

def _mesh():
    return plsc.VectorSubcoreMesh(
        core_axis_name="c", subcore_axis_name="s", num_cores=2, num_subcores=16
    )


def _worker_id():
    return lax.axis_index("s") * 2 + lax.axis_index("c")


def _gather_body(emb_hbm, tok_hbm, out_hbm,
                 idx0, idx1, rows0, rows1, idx_t, rows_t,
                 gsem0, gsem1, osem0, osem1, tsem):
    base = _worker_id() * G_ROWS
    idx = (idx0, idx1)
    rows = (rows0, rows1)
    gsem = (gsem0, gsem1)
    osem = (osem0, osem1)
    gd = [None, None]
    od = [None, None]

    def start(i):
        b = i % 2
        if od[b] is not None:
            od[b].wait()
        pltpu.sync_copy(tok_hbm.at[pl.ds(base + i * G_CHUNK, G_CHUNK)], idx[b])
        gd[b] = pltpu.async_copy(emb_hbm.at[idx[b]], rows[b], gsem[b])

    start(0)
    for i in range(G_FULL):
        if i + 1 < G_FULL:
            start(i + 1)
        b = i % 2
        gd[b].wait()
        od[b] = pltpu.async_copy(
            rows[b], out_hbm.at[pl.ds(base + i * G_CHUNK, G_CHUNK)], osem[b]
        )
    tstart = base + G_FULL * G_CHUNK
    pltpu.sync_copy(tok_hbm.at[pl.ds(tstart, G_TAIL)], idx_t)
    pltpu.async_copy(emb_hbm.at[idx_t], rows_t, tsem).wait()
    pltpu.sync_copy(rows_t, out_hbm.at[pl.ds(tstart, G_TAIL)])
    for b in range(2):
        if od[b] is not None:
            od[b].wait()


def _gather(emb, tokens):
    return pl.kernel(
        _gather_body,
        out_type=jax.ShapeDtypeStruct((N_ALL, E), jnp.float32),
        mesh=_mesh(),
        scratch_types=[
            pltpu.VMEM((G_CHUNK,), jnp.int32),
            pltpu.VMEM((G_CHUNK,), jnp.int32),
            pltpu.VMEM((G_CHUNK, E), jnp.float32),
            pltpu.VMEM((G_CHUNK, E), jnp.float32),
            pltpu.VMEM((G_TAIL,), jnp.int32),
            pltpu.VMEM((G_TAIL, E), jnp.float32),
            pltpu.SemaphoreType.DMA,
            pltpu.SemaphoreType.DMA,
            pltpu.SemaphoreType.DMA,
            pltpu.SemaphoreType.DMA,
            pltpu.SemaphoreType.DMA,
        ],
    )(emb, tokens)


def _mm_body(x_ref, w_ref, b_ref, o_ref):
    o_ref[...] = (
        jnp.dot(x_ref[...], w_ref[...], preferred_element_type=jnp.float32)
        + b_ref[...]
    )


def _combine_matmul(rows, W_c, b_c):
    blk = 256
    return pl.pallas_call(
        _mm_body,
        grid=(N_ALL // blk,),
        in_specs=[
            pl.BlockSpec((blk, E), lambda i: (i, 0)),
            pl.BlockSpec((E, E), lambda i: (0, 0)),
            pl.BlockSpec((1, E), lambda i: (0, 0)),
        ],
        out_specs=pl.BlockSpec((blk, E), lambda i: (i, 0)),
        out_shape=jax.ShapeDtypeStruct((N_ALL, E), jnp.float32),
    )(rows, W_c, b_c.reshape(1, E))


def _reduce_body(base_hbm, cp_hbm, woff_hbm, zero_hbm, out_hbm,
                 woff_v, idx_v, rows_v, root_v, acc_s, acc_m):
    wid = _worker_id()
    pltpu.sync_copy(woff_hbm, woff_v)
    seg_lo = wid * SEG_PER_W

    for side in range(2):
        crow0 = side * NC
        cp0 = side * NC
        blk = side * N_WORKERS + wid

        pltpu.sync_copy(zero_hbm, acc_s)
        pltpu.sync_copy(zero_hbm, acc_m)
        pltpu.sync_copy(
            base_hbm.at[pl.ds(ROOT0 + blk * SEG_PAD, SEG_PAD)], root_v
        )

        bvec = woff_v[pl.ds((side * N_WORKERS + wid) * 16, 16)]
        lo = bvec[0]
        hi = bvec[1]
        c0 = lo // R_CHUNK
        c1 = (hi + R_CHUNK - 1) // R_CHUNK

        def do_chunk(c, _):
            pltpu.sync_copy(cp_hbm.at[pl.ds(cp0 + c * R_CHUNK, R_CHUNK)], idx_v)
            pltpu.sync_copy(
                base_hbm.at[pl.ds(crow0 + c * R_CHUNK, R_CHUNK)], rows_v
            )

            def do_group(g, _):
                iv = idx_v[pl.ds(g * 16, 16)]
                for i in range(16):
                    ls = iv[i] - seg_lo
                    inb = jnp.logical_and(ls >= 0, ls < SEG_PER_W)
                    ls = jnp.where(inb, ls, SEG_PER_W)
                    r = g * 16 + i
                    for j in range(E // 16):
                        sl = pl.ds(j * 16, 16)
                        v = rows_v[r, sl]
                        acc_s[ls, sl] = acc_s[ls, sl] + v
                        acc_m[ls, sl] = jnp.maximum(acc_m[ls, sl], v)
                return 0

            lax.fori_loop(0, R_CHUNK // 16, do_group, 0)
            return 0

        lax.fori_loop(c0, c1, do_chunk, 0)

        def finalize(i, _):
            for j in range(E // 16):
                sl = pl.ds(j * 16, 16)
                acc_s[i, sl] = jnp.maximum(
                    root_v[i, sl] + acc_s[i, sl], acc_m[i, sl]
                )
            return 0

        lax.fori_loop(0, SEG_PER_W, finalize, 0)
        pltpu.sync_copy(acc_s, out_hbm.at[pl.ds(blk * SEG_PAD, SEG_PAD)])


def _reduce(base, cp, woff, zero):
    return pl.kernel(
        _reduce_body,
        out_type=jax.ShapeDtypeStruct((2 * N_WORKERS * SEG_PAD, E), jnp.float32),
        mesh=_mesh(),
        scratch_types=[
            pltpu.VMEM((2 * N_WORKERS * 16,), jnp.int32),
            pltpu.VMEM((R_CHUNK,), jnp.int32),
            pltpu.VMEM((R_CHUNK, E), jnp.float32),
            pltpu.VMEM((SEG_PAD, E), jnp.float32),
            pltpu.VMEM((SEG_PAD, E), jnp.float32),
            pltpu.VMEM((SEG_PAD, E), jnp.float32),
        ],
    )(base, cp, woff, zero)


def _gru_gate(gi, gh, h):
    r = jax.nn.sigmoid(gi[:, :H] + gh[:, :H])
    z = jax.nn.sigmoid(gi[:, H:2 * H] + gh[:, H:2 * H])
    n = jnp.tanh(gi[:, 2 * H:] + r * gh[:, 2 * H:])
    return (1.0 - z) * n + z * h


def _gru_body(x_ref, wif_ref, whf_ref, bif_ref, bhf_ref,
              wib_ref, whb_ref, bib_ref, bhb_ref, wo_ref, bo_ref,
              o_ref, gif_ref, gib_ref, hsf_ref):
    x = x_ref[...]
    gif_ref[...] = (
        jnp.dot(x, wif_ref[...], preferred_element_type=jnp.float32)
        + bif_ref[...]
    )
    gib_ref[...] = (
        jnp.dot(x, wib_ref[...], preferred_element_type=jnp.float32)
        + bib_ref[...]
    )
    whf = whf_ref[...]
    bhf = bhf_ref[...]

    def fstep(t, h):
        gi = gif_ref[pl.ds(t * 128, 128), :]
        gh = jnp.dot(h, whf, preferred_element_type=jnp.float32) + bhf
        h2 = _gru_gate(gi, gh, h)
        hsf_ref[pl.ds(t * 128, 128), :] = h2
        return h2

    lax.fori_loop(0, L, fstep, jnp.zeros((128, H), jnp.float32))

    whb = whb_ref[...]
    bhb = bhb_ref[...]

    def bstep(k, carry):
        h, m = carry
        t = L - 1 - k
        gi = gib_ref[pl.ds(t * 128, 128), :]
        gh = jnp.dot(h, whb, preferred_element_type=jnp.float32) + bhb
        h2 = _gru_gate(gi, gh, h)
        comb = hsf_ref[pl.ds(t * 128, 128), :] + h2
        return h2, jnp.maximum(m, comb)

    _, m = lax.fori_loop(
        0, L,
        bstep,
        (jnp.zeros((128, H), jnp.float32),
         jnp.full((128, H), -jnp.inf, jnp.float32)),
    )

    d = jnp.abs(m[:B, :] - m[B:, :])
    logits = jnp.sum(d * wo_ref[...], axis=1, keepdims=True) + bo_ref[...]
    o_ref[...] = jax.nn.sigmoid(logits)


def _gru_head(x2d, W_ih_f, W_hh_f, b_ih_f, b_hh_f,
              W_ih_b, W_hh_b, b_ih_b, b_hh_b, W_out, b_out):
    return pl.pallas_call(
        _gru_body,
        out_shape=jax.ShapeDtypeStruct((B, 1), jnp.float32),
        scratch_shapes=[
            pltpu.VMEM((L * 128, 3 * H), jnp.float32),
            pltpu.VMEM((L * 128, 3 * H), jnp.float32),
            pltpu.VMEM((L * 128, H), jnp.float32),
        ],
    )(x2d, W_ih_f, W_hh_f, b_ih_f.reshape(1, 3 * H), b_hh_f.reshape(1, 3 * H),
      W_ih_b, W_hh_b, b_ih_b.reshape(1, 3 * H), b_hh_b.reshape(1, 3 * H),
      W_out.reshape(1, H), b_out.reshape(1, 1))


def kernel(root_tokens1, child_tokens1, child_parent1,
           root_tokens2, child_tokens2, child_parent2,
           embedding, W_c, b_c,
           W_ih_f, W_hh_f, b_ih_f, b_hh_f,
           W_ih_b, W_hh_b, b_ih_b, b_hh_b,
           W_out, b_out):
    root_pad = (
        jnp.zeros((2, N_WORKERS, SEG_PAD), jnp.int32)
        .at[0, :, :SEG_PER_W].set(
            root_tokens1.astype(jnp.int32).reshape(N_WORKERS, SEG_PER_W))
        .at[1, :, :SEG_PER_W].set(
            root_tokens2.astype(jnp.int32).reshape(N_WORKERS, SEG_PER_W))
        .reshape(-1)
    )
    tokens = jnp.concatenate([
        child_tokens1.astype(jnp.int32), child_tokens2.astype(jnp.int32),
        root_pad,
    ])
    cp1 = child_parent1.astype(jnp.int32)
    cp2 = child_parent2.astype(jnp.int32)
    cp = jnp.concatenate([cp1, cp2])

    bounds = jnp.arange(0, S + 1, SEG_PER_W)
    offs1 = jnp.searchsorted(cp1, bounds).astype(jnp.int32)
    offs2 = jnp.searchsorted(cp2, bounds).astype(jnp.int32)
    woff = (
        jnp.zeros((2, N_WORKERS, 16), jnp.int32)
        .at[0, :, 0].set(offs1[:-1]).at[0, :, 1].set(offs1[1:])
        .at[1, :, 0].set(offs2[:-1]).at[1, :, 1].set(offs2[1:])
        .reshape(-1)
    )

    rows = _gather(embedding, tokens)
    base = _combine_matmul(rows, W_c, b_c)
    out_p = _reduce(base, cp, woff, jnp.zeros((SEG_PAD, E), jnp.float32))
    stmt = out_p.reshape(2, N_WORKERS, SEG_PAD, E)[:, :, :SEG_PER_W, :]

    x = jnp.transpose(stmt.reshape(2, B, L, E), (2, 0, 1, 3))
    x2d = x.reshape(L * 2 * B, E)

    return _gru_head(x2d, W_ih_f, W_hh_f, b_ih_f, b_hh_f,
                     W_ih_b, W_hh_b, b_ih_b, b_hh_b, W_out, b_out)


# speedup vs baseline: 2.0175x; 1.1378x over previous
import functools

import jax
import jax.numpy as jnp
from jax import lax
from jax.experimental import pallas as pl
from jax.experimental.pallas import tpu as pltpu
from jax.experimental.pallas import tpu_sc as plsc

S = 3200
NC = 51200
B = 64
L = 50
E = 128
H = 128

N_WORKERS = 32
SEG_PER_W = S // N_WORKERS
SEG_PAD = 104
ROOT0 = 2 * NC
N_ALL = 2 * NC + 2 * N_WORKERS * SEG_PAD
G_ROWS = N_ALL // N_WORKERS
G_CHUNK = 128
G_FULL = G_ROWS // G_CHUNK
G_TAIL = G_ROWS - G_FULL * G_CHUNK
R_CHUNK = 128

def _mesh():
    return plsc.VectorSubcoreMesh(
        core_axis_name="c", subcore_axis_name="s", num_cores=2, num_subcores=16
    )


def _worker_id():
    return lax.axis_index("s") * 2 + lax.axis_index("c")


def _gather_body(emb_hbm, tok_hbm, out_hbm,
                 idx0, idx1, rows0, rows1, idx_t, rows_t,
                 gsem0, gsem1, osem0, osem1, tsem):
    base = _worker_id() * G_ROWS
    idx = (idx0, idx1)
    rows = (rows0, rows1)
    gsem = (gsem0, gsem1)
    osem = (osem0, osem1)
    gd = [None, None]
    od = [None, None]

    def start(i):
        b = i % 2
        if od[b] is not None:
            od[b].wait()
        pltpu.sync_copy(tok_hbm.at[pl.ds(base + i * G_CHUNK, G_CHUNK)], idx[b])
        gd[b] = pltpu.async_copy(emb_hbm.at[idx[b]], rows[b], gsem[b])

    start(0)
    for i in range(G_FULL):
        if i + 1 < G_FULL:
            start(i + 1)
        b = i % 2
        gd[b].wait()
        od[b] = pltpu.async_copy(
            rows[b], out_hbm.at[pl.ds(base + i * G_CHUNK, G_CHUNK)], osem[b]
        )
    tstart = base + G_FULL * G_CHUNK
    pltpu.sync_copy(tok_hbm.at[pl.ds(tstart, G_TAIL)], idx_t)
    pltpu.async_copy(emb_hbm.at[idx_t], rows_t, tsem).wait()
    pltpu.sync_copy(rows_t, out_hbm.at[pl.ds(tstart, G_TAIL)])
    for b in range(2):
        if od[b] is not None:
            od[b].wait()


def _gather(emb, tokens):
    return pl.kernel(
        _gather_body,
        out_type=jax.ShapeDtypeStruct((N_ALL, E), jnp.float32),
        mesh=_mesh(),
        scratch_types=[
            pltpu.VMEM((G_CHUNK,), jnp.int32),
            pltpu.VMEM((G_CHUNK,), jnp.int32),
            pltpu.VMEM((G_CHUNK, E), jnp.float32),
            pltpu.VMEM((G_CHUNK, E), jnp.float32),
            pltpu.VMEM((G_TAIL,), jnp.int32),
            pltpu.VMEM((G_TAIL, E), jnp.float32),
            pltpu.SemaphoreType.DMA,
            pltpu.SemaphoreType.DMA,
            pltpu.SemaphoreType.DMA,
            pltpu.SemaphoreType.DMA,
            pltpu.SemaphoreType.DMA,
        ],
    )(emb, tokens)


def _mm_body(x_ref, w_ref, b_ref, o_ref):
    o_ref[...] = (
        jnp.dot(x_ref[...], w_ref[...], preferred_element_type=jnp.float32)
        + b_ref[...]
    )


def _combine_matmul(rows, W_c, b_c):
    blk = 256
    return pl.pallas_call(
        _mm_body,
        grid=(N_ALL // blk,),
        in_specs=[
            pl.BlockSpec((blk, E), lambda i: (i, 0)),
            pl.BlockSpec((E, E), lambda i: (0, 0)),
            pl.BlockSpec((1, E), lambda i: (0, 0)),
        ],
        out_specs=pl.BlockSpec((blk, E), lambda i: (i, 0)),
        out_shape=jax.ShapeDtypeStruct((N_ALL, E), jnp.float32),
    )(rows, W_c, b_c.reshape(1, E))


def _reduce_body(base_hbm, cp_hbm, woff_hbm, zero_hbm, out_hbm,
                 woff_v, idx_v, rows_v, root_v, acc_s, acc_m):
    wid = _worker_id()
    pltpu.sync_copy(woff_hbm, woff_v)
    seg_lo = wid * SEG_PER_W

    for side in range(2):
        crow0 = side * NC
        cp0 = side * NC
        blk = side * N_WORKERS + wid

        pltpu.sync_copy(zero_hbm, acc_s)
        pltpu.sync_copy(zero_hbm, acc_m)
        pltpu.sync_copy(
            base_hbm.at[pl.ds(ROOT0 + blk * SEG_PAD, SEG_PAD)], root_v
        )

        bvec = woff_v[pl.ds((side * N_WORKERS + wid) * 16, 16)]
        lo = bvec[0]
        hi = bvec[1]
        c0 = lo // R_CHUNK
        c1 = (hi + R_CHUNK - 1) // R_CHUNK

        def do_chunk(c, _):
            pltpu.sync_copy(cp_hbm.at[pl.ds(cp0 + c * R_CHUNK, R_CHUNK)], idx_v)
            pltpu.sync_copy(
                base_hbm.at[pl.ds(crow0 + c * R_CHUNK, R_CHUNK)], rows_v
            )

            def flush(tgt, vs, ms):
                for j in range(E // 16):
                    sl = pl.ds(j * 16, 16)
                    acc_s[tgt, sl] = acc_s[tgt, sl] + vs[j]
                    acc_m[tgt, sl] = jnp.maximum(acc_m[tgt, sl], ms[j])

            def do_group(g, _):
                iv = idx_v[pl.ds(g * 16, 16)]

                def safe_ls(i):
                    ls = iv[i] - seg_lo
                    inb = jnp.logical_and(ls >= 0, ls < SEG_PER_W)
                    return jnp.where(inb, ls, SEG_PER_W)

                prev = safe_ls(0)
                r0 = g * 16
                vs = [rows_v[r0, pl.ds(j * 16, 16)] for j in range(E // 16)]
                ms = list(vs)
                for i in range(1, 16):
                    ls = safe_ls(i)
                    same = ls == prev

                    @pl.when(jnp.logical_not(same))
                    def _(prev=prev, vs=tuple(vs), ms=tuple(ms)):
                        flush(prev, vs, ms)

                    keep = jnp.where(same, 1.0, 0.0)
                    pen = jnp.where(same, 0.0, -jnp.inf)
                    keep_v = lax.broadcast_in_dim(keep, (16,), ())
                    pen_v = lax.broadcast_in_dim(pen, (16,), ())
                    for j in range(E // 16):
                        v = rows_v[r0 + i, pl.ds(j * 16, 16)]
                        vs[j] = vs[j] * keep_v + v
                        ms[j] = jnp.maximum(ms[j] + pen_v, v)
                    prev = ls
                flush(prev, vs, ms)
                return 0

            lax.fori_loop(0, R_CHUNK // 16, do_group, 0)
            return 0

        lax.fori_loop(c0, c1, do_chunk, 0)

        def finalize(i, _):
            for j in range(E // 16):
                sl = pl.ds(j * 16, 16)
                acc_s[i, sl] = jnp.maximum(
                    root_v[i, sl] + acc_s[i, sl], acc_m[i, sl]
                )
            return 0

        lax.fori_loop(0, SEG_PER_W, finalize, 0)
        pltpu.sync_copy(acc_s, out_hbm.at[pl.ds(blk * SEG_PAD, SEG_PAD)])


def _reduce(base, cp, woff, zero):
    return pl.kernel(
        _reduce_body,
        out_type=jax.ShapeDtypeStruct((2 * N_WORKERS * SEG_PAD, E), jnp.float32),
        mesh=_mesh(),
        scratch_types=[
            pltpu.VMEM((2 * N_WORKERS * 16,), jnp.int32),
            pltpu.VMEM((R_CHUNK,), jnp.int32),
            pltpu.VMEM((R_CHUNK, E), jnp.float32),
            pltpu.VMEM((SEG_PAD, E), jnp.float32),
            pltpu.VMEM((SEG_PAD, E), jnp.float32),
            pltpu.VMEM((SEG_PAD, E), jnp.float32),
        ],
    )(base, cp, woff, zero)


def _gru_gate(gi, gh, h):
    r = jax.nn.sigmoid(gi[:, :H] + gh[:, :H])
    z = jax.nn.sigmoid(gi[:, H:2 * H] + gh[:, H:2 * H])
    n = jnp.tanh(gi[:, 2 * H:] + r * gh[:, 2 * H:])
    return (1.0 - z) * n + z * h


def _gru_body(x_ref, wif_ref, whf_ref, bif_ref, bhf_ref,
              wib_ref, whb_ref, bib_ref, bhb_ref, wo_ref, bo_ref,
              o_ref, gif_ref, gib_ref, hsf_ref):
    x = x_ref[...]
    gif_ref[...] = (
        jnp.dot(x, wif_ref[...], preferred_element_type=jnp.float32)
        + bif_ref[...]
    )
    gib_ref[...] = (
        jnp.dot(x, wib_ref[...], preferred_element_type=jnp.float32)
        + bib_ref[...]
    )
    whf = whf_ref[...]
    bhf = bhf_ref[...]

    def fstep(t, h):
        gi = gif_ref[pl.ds(t * 128, 128), :]
        gh = jnp.dot(h, whf, preferred_element_type=jnp.float32) + bhf
        h2 = _gru_gate(gi, gh, h)
        hsf_ref[pl.ds(t * 128, 128), :] = h2
        return h2

    lax.fori_loop(0, L, fstep, jnp.zeros((128, H), jnp.float32))

    whb = whb_ref[...]
    bhb = bhb_ref[...]

    def bstep(k, carry):
        h, m = carry
        t = L - 1 - k
        gi = gib_ref[pl.ds(t * 128, 128), :]
        gh = jnp.dot(h, whb, preferred_element_type=jnp.float32) + bhb
        h2 = _gru_gate(gi, gh, h)
        comb = hsf_ref[pl.ds(t * 128, 128), :] + h2
        return h2, jnp.maximum(m, comb)

    _, m = lax.fori_loop(
        0, L,
        bstep,
        (jnp.zeros((128, H), jnp.float32),
         jnp.full((128, H), -jnp.inf, jnp.float32)),
    )

    d = jnp.abs(m[:B, :] - m[B:, :])
    logits = jnp.sum(d * wo_ref[...], axis=1, keepdims=True) + bo_ref[...]
    o_ref[...] = jax.nn.sigmoid(logits)


def _gru_head(x2d, W_ih_f, W_hh_f, b_ih_f, b_hh_f,
              W_ih_b, W_hh_b, b_ih_b, b_hh_b, W_out, b_out):
    return pl.pallas_call(
        _gru_body,
        out_shape=jax.ShapeDtypeStruct((B, 1), jnp.float32),
        scratch_shapes=[
            pltpu.VMEM((L * 128, 3 * H), jnp.float32),
            pltpu.VMEM((L * 128, 3 * H), jnp.float32),
            pltpu.VMEM((L * 128, H), jnp.float32),
        ],
    )(x2d, W_ih_f, W_hh_f, b_ih_f.reshape(1, 3 * H), b_hh_f.reshape(1, 3 * H),
      W_ih_b, W_hh_b, b_ih_b.reshape(1, 3 * H), b_hh_b.reshape(1, 3 * H),
      W_out.reshape(1, H), b_out.reshape(1, 1))


def kernel(root_tokens1, child_tokens1, child_parent1,
           root_tokens2, child_tokens2, child_parent2,
           embedding, W_c, b_c,
           W_ih_f, W_hh_f, b_ih_f, b_hh_f,
           W_ih_b, W_hh_b, b_ih_b, b_hh_b,
           W_out, b_out):
    root_pad = (
        jnp.zeros((2, N_WORKERS, SEG_PAD), jnp.int32)
        .at[0, :, :SEG_PER_W].set(
            root_tokens1.astype(jnp.int32).reshape(N_WORKERS, SEG_PER_W))
        .at[1, :, :SEG_PER_W].set(
            root_tokens2.astype(jnp.int32).reshape(N_WORKERS, SEG_PER_W))
        .reshape(-1)
    )
    tokens = jnp.concatenate([
        child_tokens1.astype(jnp.int32), child_tokens2.astype(jnp.int32),
        root_pad,
    ])
    cp1 = child_parent1.astype(jnp.int32)
    cp2 = child_parent2.astype(jnp.int32)
    cp = jnp.concatenate([cp1, cp2])

    bounds = jnp.arange(0, S + 1, SEG_PER_W)
    offs1 = jnp.searchsorted(cp1, bounds).astype(jnp.int32)
    offs2 = jnp.searchsorted(cp2, bounds).astype(jnp.int32)
    woff = (
        jnp.zeros((2, N_WORKERS, 16), jnp.int32)
        .at[0, :, 0].set(offs1[:-1]).at[0, :, 1].set(offs1[1:])
        .at[1, :, 0].set(offs2[:-1]).at[1, :, 1].set(offs2[1:])
        .reshape(-1)
    )

    rows = _gather(embedding, tokens)
    base = _combine_matmul(rows, W_c, b_c)
    out_p = _reduce(base, cp, woff, jnp.zeros((SEG_PAD, E), jnp.float32))
    stmt = out_p.reshape(2, N_WORKERS, SEG_PAD, E)[:, :, :SEG_PER_W, :]

    x = jnp.transpose(stmt.reshape(2, B, L, E), (2, 0, 1, 3))
    x2d = x.reshape(L * 2 * B, E)

    return _gru_head(x2d, W_ih_f, W_hh_f, b_ih_f, b_hh_f,
                     W_ih_b, W_hh_b, b_ih_b, b_hh_b, W_out, b_out)


# speedup vs baseline: 2.0693x; 1.0257x over previous
import functools

import jax
import jax.numpy as jnp
from jax import lax
from jax.experimental import pallas as pl
from jax.experimental.pallas import tpu as pltpu
from jax.experimental.pallas import tpu_sc as plsc

S = 3200
NC = 51200
B = 64
L = 50
E = 128
H = 128

N_WORKERS = 32
SEG_PER_W = S // N_WORKERS
SEG_PAD = 104
ROOT0 = 2 * NC
N_ALL = 2 * NC + 2 * N_WORKERS * SEG_PAD
G_ROWS = N_ALL // N_WORKERS
G_CHUNK = 128
G_FULL = G_ROWS // G_CHUNK
G_TAIL = G_ROWS - G_FULL * G_CHUNK
R_CHUNK = 256

def _mesh():
    return plsc.VectorSubcoreMesh(
        core_axis_name="c", subcore_axis_name="s", num_cores=2, num_subcores=16
    )


def _worker_id():
    return lax.axis_index("s") * 2 + lax.axis_index("c")


def _gather_body(emb_hbm, tok_hbm, out_hbm,
                 idx0, idx1, rows0, rows1, idx_t, rows_t,
                 gsem0, gsem1, osem0, osem1, tsem):
    base = _worker_id() * G_ROWS
    idx = (idx0, idx1)
    rows = (rows0, rows1)
    gsem = (gsem0, gsem1)
    osem = (osem0, osem1)
    gd = [None, None]
    od = [None, None]

    def start(i):
        b = i % 2
        if od[b] is not None:
            od[b].wait()
        pltpu.sync_copy(tok_hbm.at[pl.ds(base + i * G_CHUNK, G_CHUNK)], idx[b])
        gd[b] = pltpu.async_copy(emb_hbm.at[idx[b]], rows[b], gsem[b])

    start(0)
    for i in range(G_FULL):
        if i + 1 < G_FULL:
            start(i + 1)
        b = i % 2
        gd[b].wait()
        od[b] = pltpu.async_copy(
            rows[b], out_hbm.at[pl.ds(base + i * G_CHUNK, G_CHUNK)], osem[b]
        )
    tstart = base + G_FULL * G_CHUNK
    pltpu.sync_copy(tok_hbm.at[pl.ds(tstart, G_TAIL)], idx_t)
    pltpu.async_copy(emb_hbm.at[idx_t], rows_t, tsem).wait()
    pltpu.sync_copy(rows_t, out_hbm.at[pl.ds(tstart, G_TAIL)])
    for b in range(2):
        if od[b] is not None:
            od[b].wait()


def _gather(emb, tokens):
    return pl.kernel(
        _gather_body,
        out_type=jax.ShapeDtypeStruct((N_ALL, E), jnp.float32),
        mesh=_mesh(),
        scratch_types=[
            pltpu.VMEM((G_CHUNK,), jnp.int32),
            pltpu.VMEM((G_CHUNK,), jnp.int32),
            pltpu.VMEM((G_CHUNK, E), jnp.float32),
            pltpu.VMEM((G_CHUNK, E), jnp.float32),
            pltpu.VMEM((G_TAIL,), jnp.int32),
            pltpu.VMEM((G_TAIL, E), jnp.float32),
            pltpu.SemaphoreType.DMA,
            pltpu.SemaphoreType.DMA,
            pltpu.SemaphoreType.DMA,
            pltpu.SemaphoreType.DMA,
            pltpu.SemaphoreType.DMA,
        ],
    )(emb, tokens)


def _mm_body(x_ref, w_ref, b_ref, o_ref):
    o_ref[...] = (
        jnp.dot(x_ref[...], w_ref[...], preferred_element_type=jnp.float32)
        + b_ref[...]
    )


def _combine_matmul(rows, W_c, b_c):
    blk = 256
    return pl.pallas_call(
        _mm_body,
        grid=(N_ALL // blk,),
        in_specs=[
            pl.BlockSpec((blk, E), lambda i: (i, 0)),
            pl.BlockSpec((E, E), lambda i: (0, 0)),
            pl.BlockSpec((1, E), lambda i: (0, 0)),
        ],
        out_specs=pl.BlockSpec((blk, E), lambda i: (i, 0)),
        out_shape=jax.ShapeDtypeStruct((N_ALL, E), jnp.float32),
    )(rows, W_c, b_c.reshape(1, E))


def _reduce_body(base_hbm, cp_hbm, woff_hbm, zero_hbm, out_hbm,
                 woff_v, idx_v, rows_v, root_v, acc_s, acc_m, rsem0, rsem1):
    wid = _worker_id()
    pltpu.sync_copy(woff_hbm, woff_v)
    seg_lo = wid * SEG_PER_W

    for side in range(2):
        crow0 = side * NC
        cp0 = side * NC
        blk = side * N_WORKERS + wid

        pltpu.sync_copy(zero_hbm, acc_s)
        pltpu.sync_copy(zero_hbm, acc_m)
        pltpu.sync_copy(
            base_hbm.at[pl.ds(ROOT0 + blk * SEG_PAD, SEG_PAD)], root_v
        )

        bvec = woff_v[pl.ds((side * N_WORKERS + wid) * 16, 16)]
        lo = bvec[0]
        hi = bvec[1]
        c0 = lo // R_CHUNK
        c1 = (hi + R_CHUNK - 1) // R_CHUNK

        def do_chunk(c, _):
            d1 = pltpu.async_copy(
                cp_hbm.at[pl.ds(cp0 + c * R_CHUNK, R_CHUNK)], idx_v, rsem0
            )
            d2 = pltpu.async_copy(
                base_hbm.at[pl.ds(crow0 + c * R_CHUNK, R_CHUNK)], rows_v, rsem1
            )
            d1.wait()
            d2.wait()

            def flush(tgt, vs, ms):
                for j in range(E // 16):
                    sl = pl.ds(j * 16, 16)
                    acc_s[tgt, sl] = acc_s[tgt, sl] + vs[j]
                    acc_m[tgt, sl] = jnp.maximum(acc_m[tgt, sl], ms[j])

            def do_group(g, _):
                iv = idx_v[pl.ds(g * 16, 16)]

                def safe_ls(i):
                    ls = iv[i] - seg_lo
                    inb = jnp.logical_and(ls >= 0, ls < SEG_PER_W)
                    return jnp.where(inb, ls, SEG_PER_W)

                prev = safe_ls(0)
                r0 = g * 16
                vs = [rows_v[r0, pl.ds(j * 16, 16)] for j in range(E // 16)]
                ms = list(vs)
                for i in range(1, 16):
                    ls = safe_ls(i)
                    same = ls == prev

                    @pl.when(jnp.logical_not(same))
                    def _(prev=prev, vs=tuple(vs), ms=tuple(ms)):
                        flush(prev, vs, ms)

                    keep = jnp.where(same, 1.0, 0.0)
                    pen = jnp.where(same, 0.0, -jnp.inf)
                    keep_v = lax.broadcast_in_dim(keep, (16,), ())
                    pen_v = lax.broadcast_in_dim(pen, (16,), ())
                    for j in range(E // 16):
                        v = rows_v[r0 + i, pl.ds(j * 16, 16)]
                        vs[j] = vs[j] * keep_v + v
                        ms[j] = jnp.maximum(ms[j] + pen_v, v)
                    prev = ls
                flush(prev, vs, ms)
                return 0

            lax.fori_loop(0, R_CHUNK // 16, do_group, 0)
            return 0

        lax.fori_loop(c0, c1, do_chunk, 0)

        def finalize(i, _):
            for j in range(E // 16):
                sl = pl.ds(j * 16, 16)
                acc_s[i, sl] = jnp.maximum(
                    root_v[i, sl] + acc_s[i, sl], acc_m[i, sl]
                )
            return 0

        lax.fori_loop(0, SEG_PER_W, finalize, 0)
        pltpu.sync_copy(acc_s, out_hbm.at[pl.ds(blk * SEG_PAD, SEG_PAD)])


def _reduce(base, cp, woff, zero):
    return pl.kernel(
        _reduce_body,
        out_type=jax.ShapeDtypeStruct((2 * N_WORKERS * SEG_PAD, E), jnp.float32),
        mesh=_mesh(),
        scratch_types=[
            pltpu.VMEM((2 * N_WORKERS * 16,), jnp.int32),
            pltpu.VMEM((R_CHUNK,), jnp.int32),
            pltpu.VMEM((R_CHUNK, E), jnp.float32),
            pltpu.VMEM((SEG_PAD, E), jnp.float32),
            pltpu.VMEM((SEG_PAD, E), jnp.float32),
            pltpu.VMEM((SEG_PAD, E), jnp.float32),
            pltpu.SemaphoreType.DMA,
            pltpu.SemaphoreType.DMA,
        ],
    )(base, cp, woff, zero)


def _gru_gate(gi, gh, h):
    r = jax.nn.sigmoid(gi[:, :H] + gh[:, :H])
    z = jax.nn.sigmoid(gi[:, H:2 * H] + gh[:, H:2 * H])
    n = jnp.tanh(gi[:, 2 * H:] + r * gh[:, 2 * H:])
    return (1.0 - z) * n + z * h


def _gru_body(x_ref, wif_ref, whf_ref, bif_ref, bhf_ref,
              wib_ref, whb_ref, bib_ref, bhb_ref, wo_ref, bo_ref,
              o_ref, gif_ref, gib_ref, hsf_ref):
    x = x_ref[...]
    gif_ref[...] = (
        jnp.dot(x, wif_ref[...], preferred_element_type=jnp.float32)
        + bif_ref[...]
    )
    gib_ref[...] = (
        jnp.dot(x, wib_ref[...], preferred_element_type=jnp.float32)
        + bib_ref[...]
    )
    whf = whf_ref[...]
    bhf = bhf_ref[...]

    def fstep(t, h):
        gi = gif_ref[pl.ds(t * 128, 128), :]
        gh = jnp.dot(h, whf, preferred_element_type=jnp.float32) + bhf
        h2 = _gru_gate(gi, gh, h)
        hsf_ref[pl.ds(t * 128, 128), :] = h2
        return h2

    lax.fori_loop(0, L, fstep, jnp.zeros((128, H), jnp.float32))

    whb = whb_ref[...]
    bhb = bhb_ref[...]

    def bstep(k, carry):
        h, m = carry
        t = L - 1 - k
        gi = gib_ref[pl.ds(t * 128, 128), :]
        gh = jnp.dot(h, whb, preferred_element_type=jnp.float32) + bhb
        h2 = _gru_gate(gi, gh, h)
        comb = hsf_ref[pl.ds(t * 128, 128), :] + h2
        return h2, jnp.maximum(m, comb)

    _, m = lax.fori_loop(
        0, L,
        bstep,
        (jnp.zeros((128, H), jnp.float32),
         jnp.full((128, H), -jnp.inf, jnp.float32)),
    )

    d = jnp.abs(m[:B, :] - m[B:, :])
    logits = jnp.sum(d * wo_ref[...], axis=1, keepdims=True) + bo_ref[...]
    o_ref[...] = jax.nn.sigmoid(logits)


def _gru_head(x2d, W_ih_f, W_hh_f, b_ih_f, b_hh_f,
              W_ih_b, W_hh_b, b_ih_b, b_hh_b, W_out, b_out):
    return pl.pallas_call(
        _gru_body,
        out_shape=jax.ShapeDtypeStruct((B, 1), jnp.float32),
        scratch_shapes=[
            pltpu.VMEM((L * 128, 3 * H), jnp.float32),
            pltpu.VMEM((L * 128, 3 * H), jnp.float32),
            pltpu.VMEM((L * 128, H), jnp.float32),
        ],
    )(x2d, W_ih_f, W_hh_f, b_ih_f.reshape(1, 3 * H), b_hh_f.reshape(1, 3 * H),
      W_ih_b, W_hh_b, b_ih_b.reshape(1, 3 * H), b_hh_b.reshape(1, 3 * H),
      W_out.reshape(1, H), b_out.reshape(1, 1))


def kernel(root_tokens1, child_tokens1, child_parent1,
           root_tokens2, child_tokens2, child_parent2,
           embedding, W_c, b_c,
           W_ih_f, W_hh_f, b_ih_f, b_hh_f,
           W_ih_b, W_hh_b, b_ih_b, b_hh_b,
           W_out, b_out):
    root_pad = (
        jnp.zeros((2, N_WORKERS, SEG_PAD), jnp.int32)
        .at[0, :, :SEG_PER_W].set(
            root_tokens1.astype(jnp.int32).reshape(N_WORKERS, SEG_PER_W))
        .at[1, :, :SEG_PER_W].set(
            root_tokens2.astype(jnp.int32).reshape(N_WORKERS, SEG_PER_W))
        .reshape(-1)
    )
    tokens = jnp.concatenate([
        child_tokens1.astype(jnp.int32), child_tokens2.astype(jnp.int32),
        root_pad,
    ])
    cp1 = child_parent1.astype(jnp.int32)
    cp2 = child_parent2.astype(jnp.int32)
    cp = jnp.concatenate([cp1, cp2])

    bounds = jnp.arange(0, S + 1, SEG_PER_W)
    offs1 = jnp.searchsorted(cp1, bounds).astype(jnp.int32)
    offs2 = jnp.searchsorted(cp2, bounds).astype(jnp.int32)
    woff = (
        jnp.zeros((2, N_WORKERS, 16), jnp.int32)
        .at[0, :, 0].set(offs1[:-1]).at[0, :, 1].set(offs1[1:])
        .at[1, :, 0].set(offs2[:-1]).at[1, :, 1].set(offs2[1:])
        .reshape(-1)
    )

    rows = _gather(embedding, tokens)
    base = _combine_matmul(rows, W_c, b_c)
    out_p = _reduce(base, cp, woff, jnp.zeros((SEG_PAD, E), jnp.float32))
    stmt = out_p.reshape(2, N_WORKERS, SEG_PAD, E)[:, :, :SEG_PER_W, :]

    x = jnp.transpose(stmt.reshape(2, B, L, E), (2, 0, 1, 3))
    x2d = x.reshape(L * 2 * B, E)

    return _gru_head(x2d, W_ih_f, W_hh_f, b_ih_f, b_hh_f,
                     W_ih_b, W_hh_b, b_ih_b, b_hh_b, W_out, b_out)


# speedup vs baseline: 2.0967x; 1.0133x over previous
import functools

import jax
import jax.numpy as jnp
from jax import lax
from jax.experimental import pallas as pl
from jax.experimental.pallas import tpu as pltpu
from jax.experimental.pallas import tpu_sc as plsc

S = 3200
NC = 51200
B = 64
L = 50
E = 128
H = 128

N_WORKERS = 32
SEG_PER_W = S // N_WORKERS
SEG_PAD = 104
ROOT0 = 2 * NC
N_ALL = 2 * NC + 2 * N_WORKERS * SEG_PAD
G_ROWS = N_ALL // N_WORKERS
G_CHUNK = 128
G_FULL = G_ROWS // G_CHUNK
G_TAIL = G_ROWS - G_FULL * G_CHUNK
R_CHUNK = 256

def _mesh():
    return plsc.VectorSubcoreMesh(
        core_axis_name="c", subcore_axis_name="s", num_cores=2, num_subcores=16
    )


def _worker_id():
    return lax.axis_index("s") * 2 + lax.axis_index("c")


def _gather_body(emb_hbm, tok_hbm, out_hbm,
                 idx0, idx1, rows0, rows1, idx_t, rows_t,
                 gsem0, gsem1, osem0, osem1, tsem):
    base = _worker_id() * G_ROWS
    idx = (idx0, idx1)
    rows = (rows0, rows1)
    gsem = (gsem0, gsem1)
    osem = (osem0, osem1)
    gd = [None, None]
    od = [None, None]

    def start(i):
        b = i % 2
        if od[b] is not None:
            od[b].wait()
        pltpu.sync_copy(tok_hbm.at[pl.ds(base + i * G_CHUNK, G_CHUNK)], idx[b])
        gd[b] = pltpu.async_copy(emb_hbm.at[idx[b]], rows[b], gsem[b])

    start(0)
    for i in range(G_FULL):
        if i + 1 < G_FULL:
            start(i + 1)
        b = i % 2
        gd[b].wait()
        od[b] = pltpu.async_copy(
            rows[b], out_hbm.at[pl.ds(base + i * G_CHUNK, G_CHUNK)], osem[b]
        )
    tstart = base + G_FULL * G_CHUNK
    pltpu.sync_copy(tok_hbm.at[pl.ds(tstart, G_TAIL)], idx_t)
    pltpu.async_copy(emb_hbm.at[idx_t], rows_t, tsem).wait()
    pltpu.sync_copy(rows_t, out_hbm.at[pl.ds(tstart, G_TAIL)])
    for b in range(2):
        if od[b] is not None:
            od[b].wait()


def _gather(emb, tokens):
    return pl.kernel(
        _gather_body,
        out_type=jax.ShapeDtypeStruct((N_ALL, E), jnp.float32),
        mesh=_mesh(),
        scratch_types=[
            pltpu.VMEM((G_CHUNK,), jnp.int32),
            pltpu.VMEM((G_CHUNK,), jnp.int32),
            pltpu.VMEM((G_CHUNK, E), jnp.float32),
            pltpu.VMEM((G_CHUNK, E), jnp.float32),
            pltpu.VMEM((G_TAIL,), jnp.int32),
            pltpu.VMEM((G_TAIL, E), jnp.float32),
            pltpu.SemaphoreType.DMA,
            pltpu.SemaphoreType.DMA,
            pltpu.SemaphoreType.DMA,
            pltpu.SemaphoreType.DMA,
            pltpu.SemaphoreType.DMA,
        ],
    )(emb, tokens)


def _mm_body(x_ref, w_ref, b_ref, o_ref):
    o_ref[...] = (
        jnp.dot(x_ref[...], w_ref[...], preferred_element_type=jnp.float32)
        + b_ref[...]
    )


def _combine_matmul(rows, W_c, b_c):
    blk = 256
    return pl.pallas_call(
        _mm_body,
        grid=(N_ALL // blk,),
        in_specs=[
            pl.BlockSpec((blk, E), lambda i: (i, 0)),
            pl.BlockSpec((E, E), lambda i: (0, 0)),
            pl.BlockSpec((1, E), lambda i: (0, 0)),
        ],
        out_specs=pl.BlockSpec((blk, E), lambda i: (i, 0)),
        out_shape=jax.ShapeDtypeStruct((N_ALL, E), jnp.float32),
    )(rows, W_c, b_c.reshape(1, E))


def _reduce_body(base_hbm, cp_hbm, woff_hbm, zero_hbm, out_hbm,
                 woff_v, idx_v, rows_v, root_v, acc_s, acc_m, rsem0, rsem1):
    wid = _worker_id()
    pltpu.sync_copy(woff_hbm, woff_v)
    seg_lo = wid * SEG_PER_W

    for side in range(2):
        crow0 = side * NC
        cp0 = side * NC
        blk = side * N_WORKERS + wid

        pltpu.sync_copy(zero_hbm, acc_s)
        pltpu.sync_copy(zero_hbm, acc_m)
        pltpu.sync_copy(
            base_hbm.at[pl.ds(ROOT0 + blk * SEG_PAD, SEG_PAD)], root_v
        )

        bvec = woff_v[pl.ds((side * N_WORKERS + wid) * 16, 16)]
        lo = bvec[0]
        hi = bvec[1]
        c0 = lo // R_CHUNK
        c1 = (hi + R_CHUNK - 1) // R_CHUNK

        def do_chunk(c, _):
            d1 = pltpu.async_copy(
                cp_hbm.at[pl.ds(cp0 + c * R_CHUNK, R_CHUNK)], idx_v, rsem0
            )
            d2 = pltpu.async_copy(
                base_hbm.at[pl.ds(crow0 + c * R_CHUNK, R_CHUNK)], rows_v, rsem1
            )
            d1.wait()
            d2.wait()

            def flush(tgt, vs, ms):
                for j in range(E // 16):
                    sl = pl.ds(j * 16, 16)
                    acc_s[tgt, sl] = acc_s[tgt, sl] + vs[j]
                    acc_m[tgt, sl] = jnp.maximum(acc_m[tgt, sl], ms[j])

            def do_group(g, _):
                iv = idx_v[pl.ds(g * 16, 16)]

                def safe_ls(i):
                    ls = iv[i] - seg_lo
                    inb = jnp.logical_and(ls >= 0, ls < SEG_PER_W)
                    return jnp.where(inb, ls, SEG_PER_W)

                prev = safe_ls(0)
                r0 = g * 16
                vs = [rows_v[r0, pl.ds(j * 16, 16)] for j in range(E // 16)]
                ms = list(vs)
                for i in range(1, 16):
                    ls = safe_ls(i)
                    same = ls == prev

                    @pl.when(jnp.logical_not(same))
                    def _(prev=prev, vs=tuple(vs), ms=tuple(ms)):
                        flush(prev, vs, ms)

                    keep = jnp.where(same, 1.0, 0.0)
                    pen = jnp.where(same, 0.0, -jnp.inf)
                    keep_v = lax.broadcast_in_dim(keep, (16,), ())
                    pen_v = lax.broadcast_in_dim(pen, (16,), ())
                    for j in range(E // 16):
                        v = rows_v[r0 + i, pl.ds(j * 16, 16)]
                        vs[j] = vs[j] * keep_v + v
                        ms[j] = jnp.maximum(ms[j] + pen_v, v)
                    prev = ls
                flush(prev, vs, ms)
                return 0

            lax.fori_loop(0, R_CHUNK // 16, do_group, 0)
            return 0

        lax.fori_loop(c0, c1, do_chunk, 0)

        def finalize(i, _):
            for j in range(E // 16):
                sl = pl.ds(j * 16, 16)
                acc_s[i, sl] = jnp.maximum(
                    root_v[i, sl] + acc_s[i, sl], acc_m[i, sl]
                )
            return 0

        lax.fori_loop(0, SEG_PER_W, finalize, 0)
        pltpu.sync_copy(acc_s, out_hbm.at[pl.ds(blk * SEG_PAD, SEG_PAD)])


def _reduce(base, cp, woff, zero):
    return pl.kernel(
        _reduce_body,
        out_type=jax.ShapeDtypeStruct((2 * N_WORKERS * SEG_PAD, E), jnp.float32),
        mesh=_mesh(),
        scratch_types=[
            pltpu.VMEM((2 * N_WORKERS * 16,), jnp.int32),
            pltpu.VMEM((R_CHUNK,), jnp.int32),
            pltpu.VMEM((R_CHUNK, E), jnp.float32),
            pltpu.VMEM((SEG_PAD, E), jnp.float32),
            pltpu.VMEM((SEG_PAD, E), jnp.float32),
            pltpu.VMEM((SEG_PAD, E), jnp.float32),
            pltpu.SemaphoreType.DMA,
            pltpu.SemaphoreType.DMA,
        ],
    )(base, cp, woff, zero)


def _gru_gate(gi, gh, h):
    r = jax.nn.sigmoid(gi[:, :H] + gh[:, :H])
    z = jax.nn.sigmoid(gi[:, H:2 * H] + gh[:, H:2 * H])
    n = jnp.tanh(gi[:, 2 * H:] + r * gh[:, 2 * H:])
    return (1.0 - z) * n + z * h


def _gru_body(x_ref, wif_ref, whf_ref, bif_ref, bhf_ref,
              wib_ref, whb_ref, bib_ref, bhb_ref, wo_ref, bo_ref,
              o_ref, gif_ref, gib_ref, hsf_ref, hsb_ref):
    x = x_ref[...]
    gif_ref[...] = (
        jnp.dot(x, wif_ref[...], preferred_element_type=jnp.float32)
        + bif_ref[...]
    )
    gib_ref[...] = (
        jnp.dot(x, wib_ref[...], preferred_element_type=jnp.float32)
        + bib_ref[...]
    )
    whf = whf_ref[...]
    bhf = bhf_ref[...]
    whb = whb_ref[...]
    bhb = bhb_ref[...]

    def step(k, carry):
        hf, hb = carry
        tb = L - 1 - k
        gif = gif_ref[pl.ds(k * 128, 128), :]
        gib = gib_ref[pl.ds(tb * 128, 128), :]
        ghf = jnp.dot(hf, whf, preferred_element_type=jnp.float32) + bhf
        ghb = jnp.dot(hb, whb, preferred_element_type=jnp.float32) + bhb
        hf2 = _gru_gate(gif, ghf, hf)
        hb2 = _gru_gate(gib, ghb, hb)
        hsf_ref[pl.ds(k * 128, 128), :] = hf2
        hsb_ref[pl.ds(tb * 128, 128), :] = hb2
        return hf2, hb2

    lax.fori_loop(
        0, L, step,
        (jnp.zeros((128, H), jnp.float32), jnp.zeros((128, H), jnp.float32)),
    )
    comb = hsf_ref[...] + hsb_ref[...]
    m = jnp.max(comb.reshape(L, 128, H), axis=0)

    d = jnp.abs(m[:B, :] - m[B:, :])
    logits = jnp.sum(d * wo_ref[...], axis=1, keepdims=True) + bo_ref[...]
    o_ref[...] = jax.nn.sigmoid(logits)


def _gru_head(x2d, W_ih_f, W_hh_f, b_ih_f, b_hh_f,
              W_ih_b, W_hh_b, b_ih_b, b_hh_b, W_out, b_out):
    return pl.pallas_call(
        _gru_body,
        out_shape=jax.ShapeDtypeStruct((B, 1), jnp.float32),
        scratch_shapes=[
            pltpu.VMEM((L * 128, 3 * H), jnp.float32),
            pltpu.VMEM((L * 128, 3 * H), jnp.float32),
            pltpu.VMEM((L * 128, H), jnp.float32),
            pltpu.VMEM((L * 128, H), jnp.float32),
        ],
    )(x2d, W_ih_f, W_hh_f, b_ih_f.reshape(1, 3 * H), b_hh_f.reshape(1, 3 * H),
      W_ih_b, W_hh_b, b_ih_b.reshape(1, 3 * H), b_hh_b.reshape(1, 3 * H),
      W_out.reshape(1, H), b_out.reshape(1, 1))


def kernel(root_tokens1, child_tokens1, child_parent1,
           root_tokens2, child_tokens2, child_parent2,
           embedding, W_c, b_c,
           W_ih_f, W_hh_f, b_ih_f, b_hh_f,
           W_ih_b, W_hh_b, b_ih_b, b_hh_b,
           W_out, b_out):
    root_pad = (
        jnp.zeros((2, N_WORKERS, SEG_PAD), jnp.int32)
        .at[0, :, :SEG_PER_W].set(
            root_tokens1.astype(jnp.int32).reshape(N_WORKERS, SEG_PER_W))
        .at[1, :, :SEG_PER_W].set(
            root_tokens2.astype(jnp.int32).reshape(N_WORKERS, SEG_PER_W))
        .reshape(-1)
    )
    tokens = jnp.concatenate([
        child_tokens1.astype(jnp.int32), child_tokens2.astype(jnp.int32),
        root_pad,
    ])
    cp1 = child_parent1.astype(jnp.int32)
    cp2 = child_parent2.astype(jnp.int32)
    cp = jnp.concatenate([cp1, cp2])

    bounds = jnp.arange(0, S + 1, SEG_PER_W)
    offs1 = jnp.searchsorted(cp1, bounds).astype(jnp.int32)
    offs2 = jnp.searchsorted(cp2, bounds).astype(jnp.int32)
    woff = (
        jnp.zeros((2, N_WORKERS, 16), jnp.int32)
        .at[0, :, 0].set(offs1[:-1]).at[0, :, 1].set(offs1[1:])
        .at[1, :, 0].set(offs2[:-1]).at[1, :, 1].set(offs2[1:])
        .reshape(-1)
    )

    rows = _gather(embedding, tokens)
    base = _combine_matmul(rows, W_c, b_c)
    out_p = _reduce(base, cp, woff, jnp.zeros((SEG_PAD, E), jnp.float32))
    stmt = out_p.reshape(2, N_WORKERS, SEG_PAD, E)[:, :, :SEG_PER_W, :]

    x = jnp.transpose(stmt.reshape(2, B, L, E), (2, 0, 1, 3))
    x2d = x.reshape(L * 2 * B, E)

    return _gru_head(x2d, W_ih_f, W_hh_f, b_ih_f, b_hh_f,
                     W_ih_b, W_hh_b, b_ih_b, b_hh_b, W_out, b_out)
